# baseline (speedup 1.0000x reference)
import functools

import jax
import jax.numpy as jnp
from jax import lax
from jax.experimental import pallas as pl
from jax.experimental.pallas import tpu as pltpu
from jax.experimental.pallas import tpu_sc as plsc

N = 8192
TD = 768
ID = 768
H = 512
OD = 768
E = 8
NOISE_STD = 1.0

TA = 512
TB = 256
P = 2 * N + E * TB
NB = P // TB

NW = 32
ROWS_W = P // NW
TOK_W = N // NW
GCH = 48
CCH = 32


def _proj_gate_body(xt_ref, xi_ref, wt_ref, bt_ref, wi_ref, bi_ref,
                    wg_ref, bg_ref, noise_ref, comb_ref, meta_ref, counts_ref,
                    cnt_scr):
    t = pl.program_id(0)

    @pl.when(t == 0)
    def _():
        cnt_scr[...] = jnp.zeros_like(cnt_scr)

    tp = jnp.dot(xt_ref[...], wt_ref[...], preferred_element_type=jnp.float32)
    tp = tp + bt_ref[...]
    ip = jnp.dot(xi_ref[...], wi_ref[...], preferred_element_type=jnp.float32)
    ip = ip + bi_ref[...]
    comb = jnp.concatenate([tp, ip], axis=1)
    comb_ref[...] = comb

    logits = jnp.dot(comb, wg_ref[...], preferred_element_type=jnp.float32)
    logits = logits + bg_ref[...] + noise_ref[...] * NOISE_STD

    lane = jax.lax.broadcasted_iota(jnp.int32, (TA, E), 1)
    m1 = jnp.max(logits, axis=1, keepdims=True)
    is1 = logits == m1
    idx1 = jnp.min(jnp.where(is1, lane, E), axis=1, keepdims=True)
    masked = jnp.where(lane == idx1, -jnp.inf, logits)
    m2 = jnp.max(masked, axis=1, keepdims=True)
    is2 = masked == m2
    idx2 = jnp.min(jnp.where(is2, lane, E), axis=1, keepdims=True)
    z = jnp.exp(m2 - m1)
    w1 = 1.0 / (1.0 + z)
    w2 = 1.0 - w1

    h01 = jnp.where(lane == idx1, 1.0, 0.0) + jnp.where(lane == idx2, 1.0, 0.0)
    r_iota = jax.lax.broadcasted_iota(jnp.int32, (TA, TA), 0)
    c_iota = jax.lax.broadcasted_iota(jnp.int32, (TA, TA), 1)
    tri = jnp.where(c_iota < r_iota, 1.0, 0.0)
    cumexcl = jnp.dot(tri, h01, preferred_element_type=jnp.float32)
    base = cumexcl + cnt_scr[...]
    rank1 = jnp.sum(jnp.where(lane == idx1, base, 0.0), axis=1, keepdims=True)
    rank2 = jnp.sum(jnp.where(lane == idx2, base, 0.0), axis=1, keepdims=True)
    cnt_scr[...] += jnp.sum(h01, axis=0, keepdims=True)
    counts_ref[...] = cnt_scr[...]

    idx1f = idx1.astype(jnp.float32)
    idx2f = idx2.astype(jnp.float32)
    meta_ref[...] = jnp.where(
        lane == 0, idx1f, jnp.where(
            lane == 1, idx2f, jnp.where(
                lane == 2, w1, jnp.where(
                    lane == 3, w2, jnp.where(
                        lane == 4, rank1, jnp.where(
                            lane == 5, rank2, 0.0))))))


def _grouped_ffn_body(bexp_ref, xs_ref, w1_ref, b1_ref, w2_ref, b2_ref,
                      ws_ref, y_ref, w1bf_ref, w2bf_ref):
    b = pl.program_id(0)

    @pl.when(b == 0)
    def _():
        w1bf_ref[...] = w1_ref[...].astype(jnp.bfloat16)
        w2bf_ref[...] = w2_ref[...].astype(jnp.bfloat16)

    e = bexp_ref[b]
    x = xs_ref[...].astype(jnp.bfloat16)
    row1 = jax.lax.broadcasted_iota(jnp.int32, (E, H), 0)
    b1row = jnp.sum(jnp.where(row1 == e, b1_ref[...], 0.0), axis=0,
                    keepdims=True)
    row2 = jax.lax.broadcasted_iota(jnp.int32, (E, OD), 0)
    b2row = jnp.sum(jnp.where(row2 == e, b2_ref[...], 0.0), axis=0,
                    keepdims=True)
    h = jnp.dot(x, w1bf_ref[e], preferred_element_type=jnp.float32)
    h = jnp.maximum(h + b1row, 0.0).astype(jnp.bfloat16)
    y = jnp.dot(h, w2bf_ref[e], preferred_element_type=jnp.float32)
    y = y + b2row
    y_ref[...] = y * ws_ref[...]


def _make_sort_gather():
    mesh = plsc.VectorSubcoreMesh(core_axis_name="c", subcore_axis_name="s")

    @functools.partial(
        pl.kernel, mesh=mesh,
        out_type=jax.ShapeDtypeStruct((P, 2 * H), jnp.float32),
        scratch_types=[
            pltpu.VMEM((ROWS_W,), jnp.int32),
            pltpu.VMEM((GCH, 2 * H), jnp.float32),
            pltpu.VMEM((GCH, 2 * H), jnp.float32),
            pltpu.SemaphoreType.DMA,
            pltpu.SemaphoreType.DMA,
        ],
    )
    def sort_gather(comb_hbm, gidx_hbm, xs_hbm, idx_v, buf0, buf1, sem0, sem1):
        wid = lax.axis_index("s") * 2 + lax.axis_index("c")
        base = wid * ROWS_W
        pltpu.sync_copy(gidx_hbm.at[pl.ds(base, ROWS_W)], idx_v)
        bufs = (buf0, buf1)
        sems = (sem0, sem1)
        nch = ROWS_W // GCH
        cps = []
        for c in range(nch):
            cp = pltpu.make_async_copy(
                comb_hbm.at[idx_v.at[pl.ds(c * GCH, GCH)]],
                bufs[c % 2], sems[c % 2])
            cp.start()
            cps.append(cp)
            if c >= 1:
                cps[c - 1].wait()
                pltpu.sync_copy(bufs[(c - 1) % 2],
                                xs_hbm.at[pl.ds(base + (c - 1) * GCH, GCH)])
        cps[nch - 1].wait()
        pltpu.sync_copy(bufs[(nch - 1) % 2],
                        xs_hbm.at[pl.ds(base + (nch - 1) * GCH, GCH)])

    return sort_gather


def _make_combine():
    mesh = plsc.VectorSubcoreMesh(core_axis_name="c", subcore_axis_name="s")

    @functools.partial(
        pl.kernel, mesh=mesh,
        out_type=jax.ShapeDtypeStruct((N, OD), jnp.float32),
        scratch_types=[
            pltpu.VMEM((TOK_W,), jnp.int32),
            pltpu.VMEM((TOK_W,), jnp.int32),
            pltpu.VMEM((CCH, OD), jnp.float32),
            pltpu.VMEM((CCH, OD), jnp.float32),
            pltpu.SemaphoreType.DMA,
            pltpu.SemaphoreType.DMA,
        ],
    )
    def combine(y_hbm, d1_hbm, d2_hbm, out_hbm, i1_v, i2_v, bufa, bufb,
                sema, semb):
        wid = lax.axis_index("s") * 2 + lax.axis_index("c")
        base = wid * TOK_W
        pltpu.sync_copy(d1_hbm.at[pl.ds(base, TOK_W)], i1_v)
        pltpu.sync_copy(d2_hbm.at[pl.ds(base, TOK_W)], i2_v)
        for c in range(TOK_W // CCH):
            cpa = pltpu.make_async_copy(
                y_hbm.at[i1_v.at[pl.ds(c * CCH, CCH)]], bufa, sema)
            cpb = pltpu.make_async_copy(
                y_hbm.at[i2_v.at[pl.ds(c * CCH, CCH)]], bufb, semb)
            cpa.start()
            cpb.start()
            cpa.wait()
            cpb.wait()

            def row_body(r, _):
                def col_body(j, _):
                    bufa[r, pl.ds(j * 16, 16)] = (
                        bufa[r, pl.ds(j * 16, 16)]
                        + bufb[r, pl.ds(j * 16, 16)])
                    return 0

                return lax.fori_loop(0, OD // 16, col_body, 0, unroll=8)

            lax.fori_loop(0, CCH, row_body, 0)
            pltpu.sync_copy(bufa, out_hbm.at[pl.ds(base + c * CCH, CCH)])

    return combine


def kernel(text_emb, image_emb, Wt, bt, Wi, bi, Wg, bg, W1, b1, W2, b2, noise):
    comb, meta, counts = pl.pallas_call(
        _proj_gate_body,
        grid=(N // TA,),
        in_specs=[
            pl.BlockSpec((TA, TD), lambda t: (t, 0)),
            pl.BlockSpec((TA, ID), lambda t: (t, 0)),
            pl.BlockSpec((TD, H), lambda t: (0, 0)),
            pl.BlockSpec((H,), lambda t: (0,)),
            pl.BlockSpec((ID, H), lambda t: (0, 0)),
            pl.BlockSpec((H,), lambda t: (0,)),
            pl.BlockSpec((2 * H, E), lambda t: (0, 0)),
            pl.BlockSpec((E,), lambda t: (0,)),
            pl.BlockSpec((TA, E), lambda t: (t, 0)),
        ],
        out_specs=[
            pl.BlockSpec((TA, 2 * H), lambda t: (t, 0)),
            pl.BlockSpec((TA, E), lambda t: (t, 0)),
            pl.BlockSpec((1, E), lambda t: (0, 0)),
        ],
        out_shape=[
            jax.ShapeDtypeStruct((N, 2 * H), jnp.float32),
            jax.ShapeDtypeStruct((N, E), jnp.float32),
            jax.ShapeDtypeStruct((1, E), jnp.float32),
        ],
        scratch_shapes=[pltpu.VMEM((1, E), jnp.float32)],
        compiler_params=pltpu.CompilerParams(
            dimension_semantics=("arbitrary",)),
    )(text_emb, image_emb, Wt, bt, Wi, bi, Wg, bg, noise)

    cnt = counts.reshape(E).astype(jnp.int32)
    pc = ((cnt + TB - 1) // TB) * TB
    ends = jnp.cumsum(pc)
    off = ends - pc
    e1 = meta[:, 0].astype(jnp.int32)
    e2 = meta[:, 1].astype(jnp.int32)
    w1v = meta[:, 2]
    w2v = meta[:, 3]
    d1 = off[e1] + meta[:, 4].astype(jnp.int32)
    d2 = off[e2] + meta[:, 5].astype(jnp.int32)
    tok = jnp.arange(N, dtype=jnp.int32)
    gidx = jnp.zeros((P,), jnp.int32).at[d1].set(tok).at[d2].set(tok)
    wsort = jnp.zeros((P,), jnp.float32).at[d1].set(w1v).at[d2].set(w2v)
    wsort = wsort[:, None]
    bstart = jnp.arange(NB, dtype=jnp.int32) * TB
    bexp = jnp.minimum(
        jnp.sum(bstart[:, None] >= ends[None, :], axis=1), E - 1
    ).astype(jnp.int32)

    xs = _make_sort_gather()(comb, gidx)

    ys = pl.pallas_call(
        _grouped_ffn_body,
        grid_spec=pltpu.PrefetchScalarGridSpec(
            num_scalar_prefetch=1,
            grid=(NB,),
            in_specs=[
                pl.BlockSpec((TB, 2 * H), lambda b, bexp_ref: (b, 0)),
                pl.BlockSpec((E, 2 * H, H), lambda b, bexp_ref: (0, 0, 0)),
                pl.BlockSpec((E, H), lambda b, bexp_ref: (0, 0)),
                pl.BlockSpec((E, H, OD), lambda b, bexp_ref: (0, 0, 0)),
                pl.BlockSpec((E, OD), lambda b, bexp_ref: (0, 0)),
                pl.BlockSpec((TB, 1), lambda b, bexp_ref: (b, 0)),
            ],
            out_specs=pl.BlockSpec((TB, OD), lambda b, bexp_ref: (b, 0)),
            scratch_shapes=[
                pltpu.VMEM((E, 2 * H, H), jnp.bfloat16),
                pltpu.VMEM((E, H, OD), jnp.bfloat16),
            ],
        ),
        out_shape=jax.ShapeDtypeStruct((P, OD), jnp.float32),
        compiler_params=pltpu.CompilerParams(
            dimension_semantics=("arbitrary",)),
    )(bexp, xs, W1, b1, W2, b2, wsort)

    out = _make_combine()(ys, d1, d2)
    return out


# speedup vs baseline: 2.9320x; 2.9320x over previous
import jax
import jax.numpy as jnp
from jax.experimental import pallas as pl
from jax.experimental.pallas import tpu as pltpu

N = 8192
TD = 768
ID = 768
H = 512
OD = 768
E = 8
NOISE_STD = 1.0

TA = 512
TB = 1024


def _proj_gate_body(xt_ref, xi_ref, wt_ref, bt_ref, wi_ref, bi_ref,
                    wg_ref, bg_ref, noise_ref, comb_ref, gates_ref):
    tp = jnp.dot(xt_ref[...], wt_ref[...], preferred_element_type=jnp.float32)
    tp = tp + bt_ref[...]
    ip = jnp.dot(xi_ref[...], wi_ref[...], preferred_element_type=jnp.float32)
    ip = ip + bi_ref[...]
    comb = jnp.concatenate([tp, ip], axis=1)
    comb_ref[...] = comb.astype(jnp.bfloat16)

    logits = jnp.dot(comb, wg_ref[...], preferred_element_type=jnp.float32)
    logits = logits + bg_ref[...] + noise_ref[...] * NOISE_STD

    lane = jax.lax.broadcasted_iota(jnp.int32, (TA, E), 1)
    m1 = jnp.max(logits, axis=1, keepdims=True)
    is1 = logits == m1
    idx1 = jnp.min(jnp.where(is1, lane, E), axis=1, keepdims=True)
    masked = jnp.where(lane == idx1, -jnp.inf, logits)
    m2 = jnp.max(masked, axis=1, keepdims=True)
    is2 = masked == m2
    idx2 = jnp.min(jnp.where(is2, lane, E), axis=1, keepdims=True)
    z = jnp.exp(m2 - m1)
    w1 = 1.0 / (1.0 + z)
    w2 = 1.0 - w1
    gates_ref[...] = jnp.where(lane == idx1, w1,
                               jnp.where(lane == idx2, w2, 0.0))


def _moe_body(comb_ref, gates_ref, w1_ref, b1_ref, w2_ref, b2_ref, out_ref,
              w1bf_ref, w2bf_ref, hg_ref):
    t = pl.program_id(0)

    @pl.when(t == 0)
    def _():
        w1bf_ref[...] = w1_ref[...].astype(jnp.bfloat16)
        w2bf_ref[...] = w2_ref[...].reshape(E * H, OD).astype(jnp.bfloat16)

    x = comb_ref[...]
    gates = gates_ref[...]
    lane = jax.lax.broadcasted_iota(jnp.int32, (TB, E), 1)
    EH = E // 2
    y = jnp.dot(gates, b2_ref[...], preferred_element_type=jnp.float32)
    for half in range(2):
        for k in range(EH):
            e = half * EH + k
            he = jnp.dot(x, w1bf_ref[e], preferred_element_type=jnp.float32)
            ge = jnp.sum(jnp.where(lane == e, gates, 0.0), axis=1,
                         keepdims=True)
            hg_ref[:, k * H:(k + 1) * H] = (
                jnp.maximum(he + b1_ref[e], 0.0) * ge).astype(jnp.bfloat16)
        y = y + jnp.dot(hg_ref[...],
                        w2bf_ref[pl.ds(half * EH * H, EH * H), :],
                        preferred_element_type=jnp.float32)
    out_ref[...] = y


def kernel(text_emb, image_emb, Wt, bt, Wi, bi, Wg, bg, W1, b1, W2, b2, noise):
    comb, gates = pl.pallas_call(
        _proj_gate_body,
        grid=(N // TA,),
        in_specs=[
            pl.BlockSpec((TA, TD), lambda t: (t, 0)),
            pl.BlockSpec((TA, ID), lambda t: (t, 0)),
            pl.BlockSpec((TD, H), lambda t: (0, 0)),
            pl.BlockSpec((H,), lambda t: (0,)),
            pl.BlockSpec((ID, H), lambda t: (0, 0)),
            pl.BlockSpec((H,), lambda t: (0,)),
            pl.BlockSpec((2 * H, E), lambda t: (0, 0)),
            pl.BlockSpec((E,), lambda t: (0,)),
            pl.BlockSpec((TA, E), lambda t: (t, 0)),
        ],
        out_specs=[
            pl.BlockSpec((TA, 2 * H), lambda t: (t, 0)),
            pl.BlockSpec((TA, E), lambda t: (t, 0)),
        ],
        out_shape=[
            jax.ShapeDtypeStruct((N, 2 * H), jnp.bfloat16),
            jax.ShapeDtypeStruct((N, E), jnp.float32),
        ],
        compiler_params=pltpu.CompilerParams(
            dimension_semantics=("arbitrary",)),
    )(text_emb, image_emb, Wt, bt, Wi, bi, Wg, bg, noise)

    out = pl.pallas_call(
        _moe_body,
        grid=(N // TB,),
        in_specs=[
            pl.BlockSpec((TB, 2 * H), lambda t: (t, 0)),
            pl.BlockSpec((TB, E), lambda t: (t, 0)),
            pl.BlockSpec((E, 2 * H, H), lambda t: (0, 0, 0)),
            pl.BlockSpec((E, H), lambda t: (0, 0)),
            pl.BlockSpec((E, H, OD), lambda t: (0, 0, 0)),
            pl.BlockSpec((E, OD), lambda t: (0, 0)),
        ],
        out_specs=pl.BlockSpec((TB, OD), lambda t: (t, 0)),
        out_shape=jax.ShapeDtypeStruct((N, OD), jnp.float32),
        scratch_shapes=[
            pltpu.VMEM((E, 2 * H, H), jnp.bfloat16),
            pltpu.VMEM((E * H, OD), jnp.bfloat16),
            pltpu.VMEM((TB, E * H // 2), jnp.bfloat16),
        ],
        compiler_params=pltpu.CompilerParams(
            dimension_semantics=("arbitrary",),
            vmem_limit_bytes=63 * 1024 * 1024),
    )(comb, gates, W1, b1, W2, b2)
    return out


# speedup vs baseline: 3.1931x; 1.0891x over previous
import jax
import jax.numpy as jnp
from jax.experimental import pallas as pl
from jax.experimental.pallas import tpu as pltpu

N = 8192
TD = 768
ID = 768
H = 512
OD = 768
E = 8
NOISE_STD = 1.0

T = 512


def _moe_fused_body(xt_ref, xi_ref, wt_ref, bt_ref, wi_ref, bi_ref,
                    wg_ref, bg_ref, noise_ref, w1_ref, b1_ref, w2_ref, b2_ref,
                    out_ref, w1bf_ref, w2bf_ref, hg_ref):
    t = pl.program_id(0)

    @pl.when(t == 0)
    def _():
        w1bf_ref[...] = w1_ref[...].astype(jnp.bfloat16)
        w2bf_ref[...] = w2_ref[...].reshape(E * H, OD).astype(jnp.bfloat16)

    tp = jnp.dot(xt_ref[...], wt_ref[...], preferred_element_type=jnp.float32)
    tp = tp + bt_ref[...]
    ip = jnp.dot(xi_ref[...], wi_ref[...], preferred_element_type=jnp.float32)
    ip = ip + bi_ref[...]
    comb = jnp.concatenate([tp, ip], axis=1)

    logits = jnp.dot(comb, wg_ref[...], preferred_element_type=jnp.float32)
    logits = logits + bg_ref[...] + noise_ref[...] * NOISE_STD

    lane = jax.lax.broadcasted_iota(jnp.int32, (T, E), 1)
    m1 = jnp.max(logits, axis=1, keepdims=True)
    is1 = logits == m1
    idx1 = jnp.min(jnp.where(is1, lane, E), axis=1, keepdims=True)
    masked = jnp.where(lane == idx1, -jnp.inf, logits)
    m2 = jnp.max(masked, axis=1, keepdims=True)
    is2 = masked == m2
    idx2 = jnp.min(jnp.where(is2, lane, E), axis=1, keepdims=True)
    z = jnp.exp(m2 - m1)
    w1 = 1.0 / (1.0 + z)
    w2 = 1.0 - w1
    gates = jnp.where(lane == idx1, w1, jnp.where(lane == idx2, w2, 0.0))

    x = comb.astype(jnp.bfloat16)
    EH = E // 2
    y = jnp.dot(gates, b2_ref[...], preferred_element_type=jnp.float32)
    for half in range(2):
        for k in range(EH):
            e = half * EH + k
            he = jnp.dot(x, w1bf_ref[e], preferred_element_type=jnp.float32)
            ge = jnp.sum(jnp.where(lane == e, gates, 0.0), axis=1,
                         keepdims=True)
            hg_ref[:, k * H:(k + 1) * H] = (
                jnp.maximum(he + b1_ref[e], 0.0) * ge).astype(jnp.bfloat16)
        y = y + jnp.dot(hg_ref[...],
                        w2bf_ref[pl.ds(half * EH * H, EH * H), :],
                        preferred_element_type=jnp.float32)
    out_ref[...] = y


def kernel(text_emb, image_emb, Wt, bt, Wi, bi, Wg, bg, W1, b1, W2, b2, noise):
    out = pl.pallas_call(
        _moe_fused_body,
        grid=(N // T,),
        in_specs=[
            pl.BlockSpec((T, TD), lambda t: (t, 0)),
            pl.BlockSpec((T, ID), lambda t: (t, 0)),
            pl.BlockSpec((TD, H), lambda t: (0, 0)),
            pl.BlockSpec((H,), lambda t: (0,)),
            pl.BlockSpec((ID, H), lambda t: (0, 0)),
            pl.BlockSpec((H,), lambda t: (0,)),
            pl.BlockSpec((2 * H, E), lambda t: (0, 0)),
            pl.BlockSpec((E,), lambda t: (0,)),
            pl.BlockSpec((T, E), lambda t: (t, 0)),
            pl.BlockSpec((E, 2 * H, H), lambda t: (0, 0, 0)),
            pl.BlockSpec((E, H), lambda t: (0, 0)),
            pl.BlockSpec((E, H, OD), lambda t: (0, 0, 0)),
            pl.BlockSpec((E, OD), lambda t: (0, 0)),
        ],
        out_specs=pl.BlockSpec((T, OD), lambda t: (t, 0)),
        out_shape=jax.ShapeDtypeStruct((N, OD), jnp.float32),
        scratch_shapes=[
            pltpu.VMEM((E, 2 * H, H), jnp.bfloat16),
            pltpu.VMEM((E * H, OD), jnp.bfloat16),
            pltpu.VMEM((T, E * H // 2), jnp.bfloat16),
        ],
        compiler_params=pltpu.CompilerParams(
            dimension_semantics=("arbitrary",),
            vmem_limit_bytes=63 * 1024 * 1024),
    )(text_emb, image_emb, Wt, bt, Wi, bi, Wg, bg, noise, W1, b1, W2, b2)
    return out
